# initial kernel scaffold (unmeasured)
import jax
import jax.numpy as jnp
from jax import lax
from jax.experimental import pallas as pl
from jax.experimental.pallas import tpu as pltpu

P = 16
M = 4096
N = 8192
CH = M // P
HN = N // 2


def _rows(c):
    return pl.ds(c * CH, CH)


def _ar_body(partial_ref, out_ref,
             accA, recvA, stgA, accB, recvB, stgB,
             send_semsA, recv_semsA, send_semsB, recv_semsB,
             creditA, creditB, dmaA, dmaB, outsA, outsB):
    i = lax.axis_index("i")
    right = lax.rem(i + 1, P)
    left = lax.rem(i + P - 1, P)

    colA = pl.ds(0, HN)
    colB = pl.ds(HN, HN)

    barrier = pltpu.get_barrier_semaphore()
    for nbr in (left, right):
        pl.semaphore_signal(barrier, inc=1, device_id=(nbr,),
                            device_id_type=pl.DeviceIdType.MESH)
    pl.semaphore_wait(barrier, 2)

    cp_a = pltpu.make_async_copy(partial_ref.at[_rows(i), colA], accA.at[0], dmaA)
    cp_b = pltpu.make_async_copy(partial_ref.at[_rows(i), colB], accB.at[0], dmaB)
    cp_a.start()
    cp_b.start()
    cp_a.wait()
    cp_b.wait()

    for u in range(2 * P - 2):
        slot = u % 2
        if u >= 2:
            pl.semaphore_wait(creditA, 1)
            pl.semaphore_wait(creditB, 1)

        src_a = accA.at[slot] if u <= P - 1 else recvA.at[(u - 1) % 2]
        src_b = accB.at[slot] if u <= P - 1 else recvB.at[(u - 1) % 2]
        rdma_a = pltpu.make_async_remote_copy(
            src_ref=src_a, dst_ref=recvA.at[slot],
            send_sem=send_semsA.at[slot], recv_sem=recv_semsA.at[slot],
            device_id=(right,), device_id_type=pl.DeviceIdType.MESH)
        rdma_b = pltpu.make_async_remote_copy(
            src_ref=src_b, dst_ref=recvB.at[slot],
            send_sem=send_semsB.at[slot], recv_sem=recv_semsB.at[slot],
            device_id=(left,), device_id_type=pl.DeviceIdType.MESH)
        rdma_a.start()
        rdma_b.start()

        if u < P - 1:
            ca = lax.rem(i + 2 * P - u - 1, P)
            cb = lax.rem(i + u + 1, P)
            st_a = pltpu.make_async_copy(
                partial_ref.at[_rows(ca), colA], stgA.at[slot], dmaA)
            st_b = pltpu.make_async_copy(
                partial_ref.at[_rows(cb), colB], stgB.at[slot], dmaB)
            st_a.start()
            st_b.start()
            st_a.wait()
            st_b.wait()

        rdma_a.wait()
        rdma_b.wait()

        if u < P - 1:
            accA[(u + 1) % 2] = recvA[slot] + stgA[slot]
            accB[(u + 1) % 2] = recvB[slot] + stgB[slot]
            pl.semaphore_signal(creditA, inc=1, device_id=(left,),
                                device_id_type=pl.DeviceIdType.MESH)
            pl.semaphore_signal(creditB, inc=1, device_id=(right,),
                                device_id_type=pl.DeviceIdType.MESH)
        else:
            t = u - (P - 1)
            ca = lax.rem(i + P - t, P)
            cb = lax.rem(i + t, P)
            o_a = pltpu.make_async_copy(
                recvA.at[slot], out_ref.at[_rows(ca), colA], outsA)
            o_b = pltpu.make_async_copy(
                recvB.at[slot], out_ref.at[_rows(cb), colB], outsB)
            o_a.start()
            o_b.start()
            if t == 0:
                own_a = pltpu.make_async_copy(
                    accA.at[(P - 1) % 2], out_ref.at[_rows(right), colA], dmaA)
                own_b = pltpu.make_async_copy(
                    accB.at[(P - 1) % 2], out_ref.at[_rows(left), colB], dmaB)
                own_a.start()
                own_b.start()
                own_a.wait()
                own_b.wait()
            o_a.wait()
            o_b.wait()
            if P <= u < 2 * P - 3 + 1 and u >= P:
                pl.semaphore_signal(creditA, inc=1, device_id=(left,),
                                    device_id_type=pl.DeviceIdType.MESH)
                pl.semaphore_signal(creditB, inc=1, device_id=(right,),
                                    device_id_type=pl.DeviceIdType.MESH)


def _all_reduce(partial):
    return pl.pallas_call(
        _ar_body,
        out_shape=jax.ShapeDtypeStruct((M, N), jnp.bfloat16),
        in_specs=[pl.BlockSpec(memory_space=pltpu.MemorySpace.ANY)],
        out_specs=pl.BlockSpec(memory_space=pltpu.MemorySpace.ANY),
        scratch_shapes=[
            pltpu.VMEM((2, CH, HN), jnp.bfloat16),
            pltpu.VMEM((2, CH, HN), jnp.bfloat16),
            pltpu.VMEM((2, CH, HN), jnp.bfloat16),
            pltpu.VMEM((2, CH, HN), jnp.bfloat16),
            pltpu.VMEM((2, CH, HN), jnp.bfloat16),
            pltpu.VMEM((2, CH, HN), jnp.bfloat16),
            pltpu.SemaphoreType.DMA((2,)),
            pltpu.SemaphoreType.DMA((2,)),
            pltpu.SemaphoreType.DMA((2,)),
            pltpu.SemaphoreType.DMA((2,)),
            pltpu.SemaphoreType.REGULAR,
            pltpu.SemaphoreType.REGULAR,
            pltpu.SemaphoreType.DMA,
            pltpu.SemaphoreType.DMA,
            pltpu.SemaphoreType.DMA,
            pltpu.SemaphoreType.DMA,
        ],
        compiler_params=pltpu.CompilerParams(collective_id=0),
    )(partial)


def kernel(x, w_mat):
    partial = jnp.dot(x, w_mat, preferred_element_type=jnp.float32)
    partial = partial.astype(jnp.bfloat16)

    y = _all_reduce(partial)

    yf = jnp.maximum(y.astype(jnp.float32), 0.0)
    amax = jnp.max(yf)
    scale = amax / 127.0
    q = jnp.clip(jnp.round(yf / scale), 0.0, 127.0)
    return q * scale


# baseline (device time: 971820 ns/iter reference)
import jax
import jax.numpy as jnp
from jax import lax
from jax.experimental import pallas as pl
from jax.experimental.pallas import tpu as pltpu

P = 16
M = 4096
N = 8192
CH = M // P
HN = N // 2


def _rows(c):
    return pl.ds(c * CH, CH)


def _ar_body(partial_ref, out_ref,
             accA, recvA, stgA, accB, recvB, stgB,
             send_semsA, recv_semsA, send_semsB, recv_semsB,
             creditA, creditB, dmaA, dmaB, outsA, outsB):
    i = lax.axis_index("i")
    right = lax.rem(i + 1, P)
    left = lax.rem(i + P - 1, P)

    colA = pl.ds(0, HN)
    colB = pl.ds(HN, HN)

    barrier = pltpu.get_barrier_semaphore()
    for nbr in (left, right):
        pl.semaphore_signal(barrier, inc=1, device_id=(nbr,),
                            device_id_type=pl.DeviceIdType.MESH)
    pl.semaphore_wait(barrier, 2)

    cp_a = pltpu.make_async_copy(partial_ref.at[_rows(i), colA], accA.at[0], dmaA)
    cp_b = pltpu.make_async_copy(partial_ref.at[_rows(i), colB], accB.at[0], dmaB)
    cp_a.start()
    cp_b.start()
    cp_a.wait()
    cp_b.wait()

    for u in range(2 * P - 2):
        slot = u % 2
        if u >= 2:
            pl.semaphore_wait(creditA, 1)
            pl.semaphore_wait(creditB, 1)

        src_a = accA.at[slot] if u <= P - 1 else recvA.at[(u - 1) % 2]
        src_b = accB.at[slot] if u <= P - 1 else recvB.at[(u - 1) % 2]
        rdma_a = pltpu.make_async_remote_copy(
            src_ref=src_a, dst_ref=recvA.at[slot],
            send_sem=send_semsA.at[slot], recv_sem=recv_semsA.at[slot],
            device_id=(right,), device_id_type=pl.DeviceIdType.MESH)
        rdma_b = pltpu.make_async_remote_copy(
            src_ref=src_b, dst_ref=recvB.at[slot],
            send_sem=send_semsB.at[slot], recv_sem=recv_semsB.at[slot],
            device_id=(left,), device_id_type=pl.DeviceIdType.MESH)
        rdma_a.start()
        rdma_b.start()

        if u < P - 1:
            ca = lax.rem(i + 2 * P - u - 1, P)
            cb = lax.rem(i + u + 1, P)
            st_a = pltpu.make_async_copy(
                partial_ref.at[_rows(ca), colA], stgA.at[slot], dmaA)
            st_b = pltpu.make_async_copy(
                partial_ref.at[_rows(cb), colB], stgB.at[slot], dmaB)
            st_a.start()
            st_b.start()
            st_a.wait()
            st_b.wait()

        rdma_a.wait()
        rdma_b.wait()

        if u < P - 1:
            accA[(u + 1) % 2] = recvA[slot] + stgA[slot]
            accB[(u + 1) % 2] = recvB[slot] + stgB[slot]
            pl.semaphore_signal(creditA, inc=1, device_id=(left,),
                                device_id_type=pl.DeviceIdType.MESH)
            pl.semaphore_signal(creditB, inc=1, device_id=(right,),
                                device_id_type=pl.DeviceIdType.MESH)
        else:
            t = u - (P - 1)
            ca = lax.rem(i + P - t, P)
            cb = lax.rem(i + t, P)
            o_a = pltpu.make_async_copy(
                recvA.at[slot], out_ref.at[_rows(ca), colA], outsA)
            o_b = pltpu.make_async_copy(
                recvB.at[slot], out_ref.at[_rows(cb), colB], outsB)
            o_a.start()
            o_b.start()
            if t == 0:
                own_a = pltpu.make_async_copy(
                    accA.at[(P - 1) % 2], out_ref.at[_rows(right), colA], dmaA)
                own_b = pltpu.make_async_copy(
                    accB.at[(P - 1) % 2], out_ref.at[_rows(left), colB], dmaB)
                own_a.start()
                own_b.start()
                own_a.wait()
                own_b.wait()
            o_a.wait()
            o_b.wait()
            if P <= u <= 2 * P - 4:
                pl.semaphore_signal(creditA, inc=1, device_id=(left,),
                                    device_id_type=pl.DeviceIdType.MESH)
                pl.semaphore_signal(creditB, inc=1, device_id=(right,),
                                    device_id_type=pl.DeviceIdType.MESH)


def _all_reduce(partial):
    return pl.pallas_call(
        _ar_body,
        out_shape=jax.ShapeDtypeStruct((M, N), jnp.bfloat16),
        in_specs=[pl.BlockSpec(memory_space=pl.ANY)],
        out_specs=pl.BlockSpec(memory_space=pl.ANY),
        scratch_shapes=[
            pltpu.VMEM((2, CH, HN), jnp.bfloat16),
            pltpu.VMEM((2, CH, HN), jnp.bfloat16),
            pltpu.VMEM((2, CH, HN), jnp.bfloat16),
            pltpu.VMEM((2, CH, HN), jnp.bfloat16),
            pltpu.VMEM((2, CH, HN), jnp.bfloat16),
            pltpu.VMEM((2, CH, HN), jnp.bfloat16),
            pltpu.SemaphoreType.DMA((2,)),
            pltpu.SemaphoreType.DMA((2,)),
            pltpu.SemaphoreType.DMA((2,)),
            pltpu.SemaphoreType.DMA((2,)),
            pltpu.SemaphoreType.REGULAR,
            pltpu.SemaphoreType.REGULAR,
            pltpu.SemaphoreType.DMA,
            pltpu.SemaphoreType.DMA,
            pltpu.SemaphoreType.DMA,
            pltpu.SemaphoreType.DMA,
        ],
        compiler_params=pltpu.CompilerParams(collective_id=0),
    )(partial)


def kernel(x, w_mat):
    partial = jnp.dot(x, w_mat, preferred_element_type=jnp.float32)
    partial = partial.astype(jnp.bfloat16)

    y = _all_reduce(partial)

    yf = jnp.maximum(y.astype(jnp.float32), 0.0)
    amax = jnp.max(yf)
    scale = amax / 127.0
    q = jnp.clip(jnp.round(yf / scale), 0.0, 127.0)
    return q * scale


# device time: 714522 ns/iter; 1.3601x vs baseline; 1.3601x over previous
import jax
import jax.numpy as jnp
from jax import lax
from jax.experimental import pallas as pl
from jax.experimental.pallas import tpu as pltpu

P = 16
M = 4096
N = 8192
K = 256
CH = M // P
HN = N // 2


def _rows(c):
    return pl.ds(c * CH, CH)


def _ar_body(x_ref, w_ref, outq_ref, amax_ref,
             accA, recvA, accB, recvB, qownA, qownB, qrecvA, qrecvB,
             amax_slots,
             send_semsA, recv_semsA, send_semsB, recv_semsB,
             qsend_semsA, qrecv_semsA, qsend_semsB, qrecv_semsB,
             amax_send_sems, amax_recv_sems,
             creditA, creditB, creditQA, creditQB,
             outsA, outsB):
    i = lax.axis_index("i")
    right = lax.rem(i + 1, P)
    left = lax.rem(i + P - 1, P)

    colA = pl.ds(0, HN)
    colB = pl.ds(HN, HN)

    barrier = pltpu.get_barrier_semaphore()
    for nbr in (left, right):
        pl.semaphore_signal(barrier, inc=1, device_id=(nbr,),
                            device_id_type=pl.DeviceIdType.MESH)
    pl.semaphore_wait(barrier, 2)

    def gemm(c, lo):
        a = x_ref[_rows(c), :]
        w_half = w_ref[:, lo:lo + HN]
        return jnp.dot(a, w_half,
                       preferred_element_type=jnp.float32).astype(jnp.bfloat16)

    accA[0] = gemm(i, 0)
    accB[0] = gemm(i, HN)

    for u in range(P - 1):
        slot = u % 2
        if u >= 2:
            pl.semaphore_wait(creditA, 1)
            pl.semaphore_wait(creditB, 1)
        rdma_a = pltpu.make_async_remote_copy(
            src_ref=accA.at[slot], dst_ref=recvA.at[slot],
            send_sem=send_semsA.at[slot], recv_sem=recv_semsA.at[slot],
            device_id=(right,), device_id_type=pl.DeviceIdType.MESH)
        rdma_b = pltpu.make_async_remote_copy(
            src_ref=accB.at[slot], dst_ref=recvB.at[slot],
            send_sem=send_semsB.at[slot], recv_sem=recv_semsB.at[slot],
            device_id=(left,), device_id_type=pl.DeviceIdType.MESH)
        rdma_a.start()
        rdma_b.start()

        ca = lax.rem(i + 2 * P - u - 1, P)
        cb = lax.rem(i + u + 1, P)
        gA = gemm(ca, 0)
        gB = gemm(cb, HN)

        rdma_a.wait()
        rdma_b.wait()
        accA[(u + 1) % 2] = recvA[slot] + gA
        accB[(u + 1) % 2] = recvB[slot] + gB
        if u <= P - 4:
            pl.semaphore_signal(creditA, inc=1, device_id=(left,),
                                device_id_type=pl.DeviceIdType.MESH)
            pl.semaphore_signal(creditB, inc=1, device_id=(right,),
                                device_id_type=pl.DeviceIdType.MESH)

    my_amax = jnp.maximum(
        jnp.maximum(jnp.max(accA[1].astype(jnp.float32)),
                    jnp.max(accB[1].astype(jnp.float32))),
        0.0)
    amax_slots[pl.ds(i, 1)] = jnp.full((1, 8, 128), my_amax, jnp.float32)
    for d in range(P):
        @pl.when(d != i)
        def _():
            send = pltpu.make_async_remote_copy(
                src_ref=amax_slots.at[i], dst_ref=amax_slots.at[i],
                send_sem=amax_send_sems.at[d], recv_sem=amax_recv_sems.at[i],
                device_id=(d,), device_id_type=pl.DeviceIdType.MESH)
            send.start()
    for d in range(P):
        @pl.when(d != i)
        def _():
            send = pltpu.make_async_remote_copy(
                src_ref=amax_slots.at[i], dst_ref=amax_slots.at[d],
                send_sem=amax_send_sems.at[d], recv_sem=amax_recv_sems.at[d],
                device_id=(d,), device_id_type=pl.DeviceIdType.MESH)
            send.wait_send()
            send.wait_recv()

    g = jnp.max(amax_slots[...])
    amax_ref[...] = jnp.full((8, 128), g, jnp.float32)
    inv = 127.0 / g

    def quant(chunk):
        r = jnp.maximum(chunk.astype(jnp.float32), 0.0)
        q = jnp.clip(jnp.round(r * inv), 0.0, 127.0)
        return q.astype(jnp.int8)

    qownA[...] = quant(accA[1])
    qownB[...] = quant(accB[1])

    own_a = pltpu.make_async_copy(qownA, outq_ref.at[_rows(right), colA], outsA)
    own_b = pltpu.make_async_copy(qownB, outq_ref.at[_rows(left), colB], outsB)
    own_a.start()
    own_b.start()
    own_a.wait()
    own_b.wait()

    for t in range(P - 1):
        slot = t % 2
        if t >= 2:
            pl.semaphore_wait(creditQA, 1)
            pl.semaphore_wait(creditQB, 1)
        src_a = qownA if t == 0 else qrecvA.at[(t - 1) % 2]
        src_b = qownB if t == 0 else qrecvB.at[(t - 1) % 2]
        rdma_a = pltpu.make_async_remote_copy(
            src_ref=src_a, dst_ref=qrecvA.at[slot],
            send_sem=qsend_semsA.at[slot], recv_sem=qrecv_semsA.at[slot],
            device_id=(right,), device_id_type=pl.DeviceIdType.MESH)
        rdma_b = pltpu.make_async_remote_copy(
            src_ref=src_b, dst_ref=qrecvB.at[slot],
            send_sem=qsend_semsB.at[slot], recv_sem=qrecv_semsB.at[slot],
            device_id=(left,), device_id_type=pl.DeviceIdType.MESH)
        rdma_a.start()
        rdma_b.start()
        rdma_a.wait()
        rdma_b.wait()
        if 1 <= t <= P - 3:
            pl.semaphore_signal(creditQA, inc=1, device_id=(left,),
                                device_id_type=pl.DeviceIdType.MESH)
            pl.semaphore_signal(creditQB, inc=1, device_id=(right,),
                                device_id_type=pl.DeviceIdType.MESH)
        ca = lax.rem(i + P - t, P)
        cb = lax.rem(i + t, P)
        o_a = pltpu.make_async_copy(
            qrecvA.at[slot], outq_ref.at[_rows(ca), colA], outsA)
        o_b = pltpu.make_async_copy(
            qrecvB.at[slot], outq_ref.at[_rows(cb), colB], outsB)
        o_a.start()
        o_b.start()
        o_a.wait()
        o_b.wait()


def _fused_gemm_ar(x_bf, w_bf):
    return pl.pallas_call(
        _ar_body,
        out_shape=[
            jax.ShapeDtypeStruct((M, N), jnp.int8),
            jax.ShapeDtypeStruct((8, 128), jnp.float32),
        ],
        in_specs=[
            pl.BlockSpec(memory_space=pltpu.MemorySpace.VMEM),
            pl.BlockSpec(memory_space=pltpu.MemorySpace.VMEM),
        ],
        out_specs=[
            pl.BlockSpec(memory_space=pl.ANY),
            pl.BlockSpec(memory_space=pltpu.MemorySpace.VMEM),
        ],
        scratch_shapes=[
            pltpu.VMEM((2, CH, HN), jnp.bfloat16),
            pltpu.VMEM((2, CH, HN), jnp.bfloat16),
            pltpu.VMEM((2, CH, HN), jnp.bfloat16),
            pltpu.VMEM((2, CH, HN), jnp.bfloat16),
            pltpu.VMEM((CH, HN), jnp.int8),
            pltpu.VMEM((CH, HN), jnp.int8),
            pltpu.VMEM((2, CH, HN), jnp.int8),
            pltpu.VMEM((2, CH, HN), jnp.int8),
            pltpu.VMEM((P, 8, 128), jnp.float32),
            pltpu.SemaphoreType.DMA((2,)),
            pltpu.SemaphoreType.DMA((2,)),
            pltpu.SemaphoreType.DMA((2,)),
            pltpu.SemaphoreType.DMA((2,)),
            pltpu.SemaphoreType.DMA((2,)),
            pltpu.SemaphoreType.DMA((2,)),
            pltpu.SemaphoreType.DMA((2,)),
            pltpu.SemaphoreType.DMA((2,)),
            pltpu.SemaphoreType.DMA((P,)),
            pltpu.SemaphoreType.DMA((P,)),
            pltpu.SemaphoreType.REGULAR,
            pltpu.SemaphoreType.REGULAR,
            pltpu.SemaphoreType.REGULAR,
            pltpu.SemaphoreType.REGULAR,
            pltpu.SemaphoreType.DMA,
            pltpu.SemaphoreType.DMA,
        ],
        compiler_params=pltpu.CompilerParams(collective_id=0),
    )(x_bf, w_bf)


def kernel(x, w_mat):
    x_bf = x.astype(jnp.bfloat16)
    w_bf = w_mat.astype(jnp.bfloat16)
    q, amax = _fused_gemm_ar(x_bf, w_bf)
    scale = amax[0, 0] / 127.0
    return q.astype(jnp.float32) * scale


# device time: 607223 ns/iter; 1.6004x vs baseline; 1.1767x over previous
import jax
import jax.numpy as jnp
from jax import lax
from jax.experimental import pallas as pl
from jax.experimental.pallas import tpu as pltpu

P = 16
M = 4096
N = 8192
K = 256
CH = M // P
NR = 4
QN = N // NR
RIGHT = (True, True, False, False)


def _rows(c):
    return pl.ds(c * CH, CH)


def _ar_body(x_ref, w_ref, out_ref, *s):
    acc = s[0:4]
    recv = s[4:8]
    deq = s[8:12]
    qown = s[12:16]
    qrecv = s[16:20]
    amax_slots = s[20]
    send_sems = s[21:25]
    recv_sems = s[25:29]
    qsend_sems = s[29:33]
    qrecv_sems = s[33:37]
    outs = s[37:41]
    amax_send_sems = s[41]
    amax_recv_sems = s[42]
    credit = s[43:47]
    creditq = s[47:51]

    i = lax.axis_index("i")
    right = lax.rem(i + 1, P)
    left = lax.rem(i + P - 1, P)

    def nbr_out(r):
        return right if RIGHT[r] else left

    def nbr_in(r):
        return left if RIGHT[r] else right

    def rs_add_chunk(r, u):
        if RIGHT[r]:
            return lax.rem(i + 2 * P - u - 1, P)
        return lax.rem(i + u + 1, P)

    def ag_chunk(r, t):
        if RIGHT[r]:
            return lax.rem(i + P - t, P)
        return lax.rem(i + t, P)

    def own_chunk(r):
        return right if RIGHT[r] else left

    def col(r):
        return pl.ds(r * QN, QN)

    def gemm(c, r):
        a = x_ref[_rows(c), :]
        w_q = w_ref[:, r * QN:(r + 1) * QN]
        return jnp.dot(a, w_q,
                       preferred_element_type=jnp.float32).astype(jnp.bfloat16)

    def mk_rs(r, u):
        slot = u % 2
        return pltpu.make_async_remote_copy(
            src_ref=acc[r].at[slot], dst_ref=recv[r].at[slot],
            send_sem=send_sems[r].at[slot], recv_sem=recv_sems[r].at[slot],
            device_id=(nbr_out(r),), device_id_type=pl.DeviceIdType.MESH)

    def mk_ag(r, t):
        slot = t % 2
        src = qown[r] if t == 0 else qrecv[r].at[(t - 1) % 2]
        return pltpu.make_async_remote_copy(
            src_ref=src, dst_ref=qrecv[r].at[slot],
            send_sem=qsend_sems[r].at[slot], recv_sem=qrecv_sems[r].at[slot],
            device_id=(nbr_out(r),), device_id_type=pl.DeviceIdType.MESH)

    def mk_store(r, t):
        if t < 0:
            return pltpu.make_async_copy(
                acc[r].at[0], out_ref.at[_rows(own_chunk(r)), col(r)],
                outs[r].at[0])
        return pltpu.make_async_copy(
            deq[r].at[t % 2], out_ref.at[_rows(ag_chunk(r, t)), col(r)],
            outs[r].at[t % 2])

    def signal(sem, target):
        pl.semaphore_signal(sem, inc=1, device_id=(target,),
                            device_id_type=pl.DeviceIdType.MESH)

    barrier = pltpu.get_barrier_semaphore()
    for nbr in (left, right):
        signal(barrier, nbr)
    pl.semaphore_wait(barrier, 2)

    for r in range(NR):
        acc[r][0] = gemm(i, r)
    for r in range(NR):
        mk_rs(r, 0).start()

    for u in range(P - 1):
        for r in range(NR):
            g = gemm(rs_add_chunk(r, u), r)
            mk_rs(r, u).wait()
            acc[r][(u + 1) % 2] = recv[r][u % 2] + g
            if u <= P - 4:
                signal(credit[r], nbr_in(r))
            if u + 1 <= P - 2:
                if u + 1 >= 2:
                    pl.semaphore_wait(credit[r], 1)
                mk_rs(r, u + 1).start()

    m01 = jnp.maximum(jnp.max(acc[0][1].astype(jnp.float32)),
                      jnp.max(acc[1][1].astype(jnp.float32)))
    m23 = jnp.maximum(jnp.max(acc[2][1].astype(jnp.float32)),
                      jnp.max(acc[3][1].astype(jnp.float32)))
    my_amax = jnp.maximum(jnp.maximum(m01, m23), 0.0)
    amax_slots[pl.ds(i, 1)] = jnp.full((1, 8, 128), my_amax, jnp.float32)
    for d in range(P):
        @pl.when(d != i)
        def _():
            pltpu.make_async_remote_copy(
                src_ref=amax_slots.at[i], dst_ref=amax_slots.at[i],
                send_sem=amax_send_sems.at[d], recv_sem=amax_recv_sems.at[i],
                device_id=(d,), device_id_type=pl.DeviceIdType.MESH).start()
    for d in range(P):
        @pl.when(d != i)
        def _():
            w = pltpu.make_async_remote_copy(
                src_ref=amax_slots.at[i], dst_ref=amax_slots.at[d],
                send_sem=amax_send_sems.at[d], recv_sem=amax_recv_sems.at[d],
                device_id=(d,), device_id_type=pl.DeviceIdType.MESH)
            w.wait_send()
            w.wait_recv()

    g_amax = jnp.max(amax_slots[...])
    scale = g_amax / 127.0
    inv = 127.0 / g_amax

    for r in range(NR):
        q = jnp.clip(jnp.round(
            jnp.maximum(acc[r][1].astype(jnp.float32), 0.0) * inv),
            0.0, 127.0).astype(jnp.int8)
        qown[r][...] = q
        acc[r][0] = (q.astype(jnp.float32) * scale).astype(jnp.bfloat16)
    for r in range(NR):
        mk_ag(r, 0).start()
        mk_store(r, -1).start()

    for t in range(P - 1):
        for r in range(NR):
            mk_ag(r, t).wait()
            if 1 <= t <= P - 3:
                signal(creditq[r], nbr_in(r))
            d_val = (qrecv[r][t % 2].astype(jnp.float32)
                     * scale).astype(jnp.bfloat16)
            if t == 0:
                mk_store(r, -1).wait()
            elif t >= 2:
                mk_store(r, t - 2).wait()
            deq[r][t % 2] = d_val
            mk_store(r, t).start()
            if t + 1 <= P - 2:
                if t + 1 >= 2:
                    pl.semaphore_wait(creditq[r], 1)
                mk_ag(r, t + 1).start()

    for r in range(NR):
        mk_store(r, P - 3).wait()
        mk_store(r, P - 2).wait()


def _fused_gemm_ar(x_bf, w_bf):
    ring_vmem = lambda dt, lead: [pltpu.VMEM(lead + (CH, QN), dt)
                                  for _ in range(NR)]
    return pl.pallas_call(
        _ar_body,
        out_shape=jax.ShapeDtypeStruct((M, N), jnp.bfloat16),
        in_specs=[
            pl.BlockSpec(memory_space=pltpu.MemorySpace.VMEM),
            pl.BlockSpec(memory_space=pltpu.MemorySpace.VMEM),
        ],
        out_specs=pl.BlockSpec(memory_space=pl.ANY),
        scratch_shapes=(
            ring_vmem(jnp.bfloat16, (2,))
            + ring_vmem(jnp.bfloat16, (2,))
            + ring_vmem(jnp.bfloat16, (2,))
            + ring_vmem(jnp.int8, ())
            + ring_vmem(jnp.int8, (2,))
            + [pltpu.VMEM((P, 8, 128), jnp.float32)]
            + [pltpu.SemaphoreType.DMA((2,)) for _ in range(4 * NR)]
            + [pltpu.SemaphoreType.DMA((2,)) for _ in range(NR)]
            + [pltpu.SemaphoreType.DMA((P,)) for _ in range(2)]
            + [pltpu.SemaphoreType.REGULAR for _ in range(2 * NR)]
        ),
        compiler_params=pltpu.CompilerParams(
            collective_id=0, vmem_limit_bytes=64 * 1024 * 1024),
    )(x_bf, w_bf)


def kernel(x, w_mat):
    x_bf = x.astype(jnp.bfloat16)
    w_bf = w_mat.astype(jnp.bfloat16)
    return _fused_gemm_ar(x_bf, w_bf)
